# baseline (speedup 1.0000x reference)
import functools

import jax
import jax.numpy as jnp
from jax import lax
from jax.experimental import pallas as pl
from jax.experimental.pallas import tpu as pltpu
from jax.experimental.pallas import tpu_sc as plsc

_N = 10000
_E = 160000
_NW = 32
_CHUNK = 128
_TCHUNKS = 1280
_EPAD = _TCHUNKS * _CHUNK
_K0 = 60
_K1 = 20
_KMAX = max(_K0, _K1)
_NPAD = 10112
_RT = _NPAD // 16


def _make_agg(F, nbuf):
    mesh = plsc.VectorSubcoreMesh(core_axis_name="c", subcore_axis_name="s")

    @functools.partial(
        pl.kernel,
        mesh=mesh,
        compiler_params=pltpu.CompilerParams(use_tc_tiling_on_sc=False),
        out_type=jax.ShapeDtypeStruct((2, _NPAD, F), jnp.float32),
        scratch_types=(
            [pltpu.VMEM((_KMAX, 2, _CHUNK), jnp.int32)]
            + [pltpu.VMEM((_CHUNK, F), jnp.float32)] * nbuf
            + [pltpu.VMEM_SHARED((_NPAD, F), jnp.float32)]
            + [pltpu.SemaphoreType.DMA] * (1 + 2 * nbuf)
        ),
    )
    def agg(h_hbm, idx_hbm, zero_hbm, out_hbm, *scr):
        idx_v = scr[0]
        msgs = scr[1:1 + nbuf]
        acc_sh = scr[1 + nbuf]
        semi = scr[2 + nbuf]
        sgs = scr[3 + nbuf:3 + 2 * nbuf]
        sss = scr[3 + 2 * nbuf:3 + 3 * nbuf]
        cid = lax.axis_index("c")
        sid = lax.axis_index("s")
        n = _K0 + (_K1 - _K0) * cid
        cbase = cid * (16 * _K0) + sid * n

        idx_cp = pltpu.make_async_copy(
            idx_hbm.at[pl.ds(cbase, _KMAX)], idx_v, semi)
        idx_cp.start()
        z0 = sid * _RT
        pltpu.sync_copy(zero_hbm, acc_sh.at[pl.ds(z0, _RT), :])
        idx_cp.wait()
        plsc.subcore_barrier()

        def start_g(c, b):
            pltpu.async_copy(h_hbm.at[idx_v.at[c, 0]], msgs[b], sgs[b])

        def wait_g(c, b):
            pltpu.make_async_copy(h_hbm.at[idx_v.at[c, 0]], msgs[b],
                                  sgs[b]).wait()

        def start_s(c, b):
            pltpu.async_copy(msgs[b], acc_sh.at[idx_v.at[c, 1]], sss[b],
                             add=True)

        def wait_s(c, b):
            pltpu.make_async_copy(msgs[b], acc_sh.at[idx_v.at[c, 1]],
                                  sss[b]).wait()

        for c in range(nbuf - 1):
            start_g(c, c)

        def group(g, carry):
            for j in range(nbuf):
                u = g * nbuf + j
                bprev = (j - 1) % nbuf
                wait_g(u, j)
                start_s(u, j)

                @pl.when(u >= 1)
                def _():
                    wait_s(u - 1, bprev)

                @pl.when(u + nbuf - 1 < n)
                def _():
                    start_g(u + nbuf - 1, bprev)

            return carry

        lax.fori_loop(0, n // nbuf, group, 0)
        wait_s(n - 1, nbuf - 1)
        plsc.subcore_barrier()

        pltpu.sync_copy(acc_sh.at[pl.ds(z0, _RT), :],
                        out_hbm.at[cid, pl.ds(z0, _RT), :])

    return agg


_AGG = {F: _make_agg(F, nbuf=(2 if F == 128 else 4)) for F in (16, 32, 64, 128)}


_INV_SQRT2 = 0.7071067811865476


def _gelu(x):
    return 0.5 * x * (1.0 + lax.erf(x * _INV_SQRT2))


def _bn(u, g, bt):
    m = jnp.mean(u, axis=0, keepdims=True)
    d = u - m
    v = jnp.mean(d * d, axis=0, keepdims=True)
    return d * lax.rsqrt(v + 1e-5) * g + bt


def _psum(p_ref):
    return p_ref[0, :_N, :] + p_ref[1, :_N, :]


def _dis_body(p_ref, o_ref):
    cnt = p_ref[0, :_N, 0:1] + p_ref[1, :_N, 0:1]
    o_ref[...] = lax.rsqrt(cnt + 1.0)


def _tc1_body(x_ref, w_ref, dis_ref, o_ref):
    t = jnp.dot(x_ref[...], w_ref[...], preferred_element_type=jnp.float32)
    o_ref[...] = t * dis_ref[...]


def _tc2_body(p_ref, s_ref, dis_ref, b_ref, g_ref, bt_ref, o_ref):
    a = (_psum(p_ref) + s_ref[...]) * dis_ref[...]
    h = _gelu(_bn(a + b_ref[...], g_ref[...], bt_ref[...]))
    o_ref[...] = h * dis_ref[...]


def _tcm_body(p_ref, s_ref, dis_ref, w_ref, b_ref, g_ref, bt_ref, o_ref):
    a = (_psum(p_ref) + s_ref[...]) * dis_ref[...]
    k = w_ref.shape[0]
    u = jnp.dot(a[:, :k], w_ref[...],
                preferred_element_type=jnp.float32) + b_ref[...]
    h = _gelu(_bn(u, g_ref[...], bt_ref[...]))
    o_ref[...] = h * dis_ref[...]


def _tc6_body(p_ref, s_ref, dis_ref, w_ref, b_ref, g_ref, bt_ref,
              oa_ref, ob_ref):
    a = (_psum(p_ref) + s_ref[...]) * dis_ref[...]
    u = jnp.dot(a, w_ref[...], preferred_element_type=jnp.float32) + b_ref[...]
    h = _gelu(_bn(u, g_ref[...], bt_ref[...])) * dis_ref[...]
    oa_ref[...] = h[:, :128]
    ob_ref[...] = h[:, 128:]


_RB = 2000


def _comb6_body(pa_ref, pb_ref, sa_ref, sb_ref, dis_ref, o_ref):
    aa = pa_ref[0] + pa_ref[1] + sa_ref[...]
    ab = pb_ref[0] + pb_ref[1] + sb_ref[...]
    o_ref[...] = jnp.concatenate([aa, ab], axis=1) * dis_ref[...]


def _comb6(pa, pb, sa, sb, dis):
    return pl.pallas_call(
        _comb6_body,
        grid=(_N // _RB,),
        in_specs=[
            pl.BlockSpec((2, _RB, 128), lambda i: (0, i, 0)),
            pl.BlockSpec((2, _RB, 128), lambda i: (0, i, 0)),
            pl.BlockSpec((_RB, 128), lambda i: (i, 0)),
            pl.BlockSpec((_RB, 128), lambda i: (i, 0)),
            pl.BlockSpec((_RB, 1), lambda i: (i, 0)),
        ],
        out_specs=pl.BlockSpec((_RB, 256), lambda i: (i, 0)),
        out_shape=jax.ShapeDtypeStruct((_N, 256), jnp.float32),
    )(pa, pb, sa, sb, dis)


def _tc7_body(a_ref, dis_ref, w6_ref, b6_ref, g6_ref, bt6_ref, w7_ref, o_ref):
    j = pl.program_id(0)
    u = jnp.dot(a_ref[...], w6_ref[...],
                preferred_element_type=jnp.float32) + b6_ref[...]
    h = _gelu(_bn(u, g6_ref[...], bt6_ref[...]))
    pt = jnp.dot(h, w7_ref[...], preferred_element_type=jnp.float32)

    @pl.when(j == 0)
    def _():
        o_ref[...] = pt

    @pl.when(j > 0)
    def _():
        o_ref[...] = o_ref[...] + pt

    @pl.when(j == 3)
    def _():
        o_ref[...] = o_ref[...] * dis_ref[...]


def _tc7(a6, dis, w6, b6, g6, bt6, w7p):
    return pl.pallas_call(
        _tc7_body,
        grid=(4,),
        in_specs=[
            pl.BlockSpec((_N, 256), lambda j: (0, 0)),
            pl.BlockSpec((_N, 1), lambda j: (0, 0)),
            pl.BlockSpec((256, 128), lambda j: (0, j)),
            pl.BlockSpec((1, 128), lambda j: (0, j)),
            pl.BlockSpec((1, 128), lambda j: (0, j)),
            pl.BlockSpec((1, 128), lambda j: (0, j)),
            pl.BlockSpec((128, 64), lambda j: (j, 0)),
        ],
        out_specs=pl.BlockSpec((_N, 64), lambda j: (0, 0)),
        out_shape=jax.ShapeDtypeStruct((_N, 64), jnp.float32),
    )(a6, dis, w6, b6, g6, bt6, w7p)


def _tc8_body(p_ref, s_ref, dis_ref, b_ref, o_ref):
    a = (_psum(p_ref) + s_ref[...]) * dis_ref[...]
    u = a[:, :40] + b_ref[...]
    mx = jnp.max(u, axis=1, keepdims=True)
    sh = u - mx
    lse = jnp.log(jnp.sum(jnp.exp(sh), axis=1, keepdims=True))
    o_ref[...] = sh - lse


def _tc(body, out_shape):
    return pl.pallas_call(body, out_shape=out_shape)


def kernel(x, edge_index, W1, b1, W2, b2, W3, b3, W4, b4, W5, b5, W6, b6,
           W7, b7, g1, bt1, g2, bt2, g3, bt3, g4, bt4, g5, bt5, g6, bt6):
    f32 = jnp.float32
    src = edge_index[0].astype(jnp.int32)
    dst = edge_index[1].astype(jnp.int32)
    pad = _EPAD - _E
    src_p = jnp.concatenate([src, jnp.zeros((pad,), jnp.int32)])
    dst_p = jnp.concatenate([dst, jnp.full((pad,), _N, jnp.int32)])
    idx_p = jnp.stack([src_p.reshape(_TCHUNKS, _CHUNK),
                       dst_p.reshape(_TCHUNKS, _CHUNK)], axis=1)

    zeros = {F: jnp.zeros((_RT, F), f32) for F in (16, 32, 64, 128)}
    ones16 = jnp.ones((_N, 16), f32)

    def agg(h, F):
        return _AGG[F](h, idx_p, zeros[F])

    row = lambda v: v.reshape(1, -1)

    p0 = agg(ones16, 16)
    dis = _tc(_dis_body, jax.ShapeDtypeStruct((_N, 1), f32))(p0)

    s1 = _tc(_tc1_body, jax.ShapeDtypeStruct((_N, 16), f32))(x, W1, dis)
    p1 = agg(s1, 16)
    s2 = _tc(_tc2_body, jax.ShapeDtypeStruct((_N, 16), f32))(
        p1, s1, dis, row(b1), row(g1), row(bt1))
    p2 = agg(s2, 16)
    s3 = _tc(_tcm_body, jax.ShapeDtypeStruct((_N, 32), f32))(
        p2, s2, dis, W2, row(b2), row(g2), row(bt2))
    p3 = agg(s3, 32)
    s4 = _tc(_tcm_body, jax.ShapeDtypeStruct((_N, 64), f32))(
        p3, s3, dis, W3, row(b3), row(g3), row(bt3))
    p4 = agg(s4, 64)
    s5 = _tc(_tcm_body, jax.ShapeDtypeStruct((_N, 128), f32))(
        p4, s4, dis, W4, row(b4), row(g4), row(bt4))
    p5 = agg(s5, 128)
    s6a, s6b = _tc(_tc6_body, (jax.ShapeDtypeStruct((_N, 128), f32),
                               jax.ShapeDtypeStruct((_N, 128), f32)))(
        p5, s5, dis, W5, row(b5), row(g5), row(bt5))
    p6a = _AGG[128](s6a, idx_p, zeros[128])
    p6b = _AGG[128](s6b, idx_p, zeros[128])
    w7p = jnp.concatenate([W7, jnp.zeros((512, 24), f32)], axis=1)
    a6 = _comb6(p6a, p6b, s6a, s6b, dis)
    s7 = _tc7(a6, dis, W6, row(b6), row(g6), row(bt6), w7p)
    p7 = agg(s7, 64)
    out = _tc(_tc8_body, jax.ShapeDtypeStruct((_N, 40), f32))(
        p7, s7, dis, row(b7))
    return out


# speedup vs baseline: 1.0321x; 1.0321x over previous
import functools

import jax
import jax.numpy as jnp
from jax import lax
from jax.experimental import pallas as pl
from jax.experimental.pallas import tpu as pltpu
from jax.experimental.pallas import tpu_sc as plsc

_N = 10000
_E = 160000
_NW = 32
_CHUNK = 128
_TCHUNKS = 1280
_EPAD = _TCHUNKS * _CHUNK
_K0 = 60
_K1 = 20
_KMAX = max(_K0, _K1)
_NPAD = 10112
_RT = _NPAD // 16


def _make_agg(F, nbuf):
    mesh = plsc.VectorSubcoreMesh(core_axis_name="c", subcore_axis_name="s")

    @functools.partial(
        pl.kernel,
        mesh=mesh,
        compiler_params=pltpu.CompilerParams(use_tc_tiling_on_sc=False),
        out_type=jax.ShapeDtypeStruct((2, _NPAD, F), jnp.float32),
        scratch_types=(
            [pltpu.VMEM((_KMAX, 2, _CHUNK), jnp.int32)]
            + [pltpu.VMEM((_CHUNK, F), jnp.float32)] * nbuf
            + [pltpu.VMEM_SHARED((_NPAD, F), jnp.float32)]
            + [pltpu.SemaphoreType.DMA] * (1 + 2 * nbuf)
        ),
    )
    def agg(h_hbm, idx_hbm, out_hbm, *scr):
        idx_v = scr[0]
        msgs = scr[1:1 + nbuf]
        acc_sh = scr[1 + nbuf]
        semi = scr[2 + nbuf]
        sgs = scr[3 + nbuf:3 + 2 * nbuf]
        sss = scr[3 + 2 * nbuf:3 + 3 * nbuf]
        cid = lax.axis_index("c")
        sid = lax.axis_index("s")
        n = _K0 + (_K1 - _K0) * cid
        cbase = cid * (16 * _K0) + sid * n

        idx_cp = pltpu.make_async_copy(
            idx_hbm.at[pl.ds(cbase, _KMAX)], idx_v, semi)
        idx_cp.start()
        zbuf = msgs[0]

        def zrow(i, carry):
            for c in range(F // 16):
                zbuf[i, pl.ds(c * 16, 16)] = jnp.zeros((16,), jnp.float32)
            return carry

        lax.fori_loop(0, _CHUNK, zrow, 0)
        z0 = sid * _RT
        for b in range(_RT // _CHUNK):
            pltpu.sync_copy(zbuf, acc_sh.at[pl.ds(z0 + b * _CHUNK, _CHUNK), :])
        _rem = _RT % _CHUNK
        pltpu.sync_copy(zbuf.at[pl.ds(0, _rem), :],
                        acc_sh.at[pl.ds(z0 + _RT - _rem, _rem), :])
        idx_cp.wait()
        plsc.subcore_barrier()

        def start_g(c, b):
            pltpu.async_copy(h_hbm.at[idx_v.at[c, 0]], msgs[b], sgs[b])

        def wait_g(c, b):
            pltpu.make_async_copy(h_hbm.at[idx_v.at[c, 0]], msgs[b],
                                  sgs[b]).wait()

        def start_s(c, b):
            pltpu.async_copy(msgs[b], acc_sh.at[idx_v.at[c, 1]], sss[b],
                             add=True)

        def wait_s(c, b):
            pltpu.make_async_copy(msgs[b], acc_sh.at[idx_v.at[c, 1]],
                                  sss[b]).wait()

        for c in range(nbuf - 1):
            start_g(c, c)

        def group(g, carry):
            for j in range(nbuf):
                u = g * nbuf + j
                bprev = (j - 1) % nbuf
                wait_g(u, j)
                start_s(u, j)

                @pl.when(u >= 1)
                def _():
                    wait_s(u - 1, bprev)

                @pl.when(u + nbuf - 1 < n)
                def _():
                    start_g(u + nbuf - 1, bprev)

            return carry

        lax.fori_loop(0, n // nbuf, group, 0)
        wait_s(n - 1, nbuf - 1)
        plsc.subcore_barrier()

        pltpu.sync_copy(acc_sh.at[pl.ds(z0, _RT), :],
                        out_hbm.at[cid, pl.ds(z0, _RT), :])

    return agg


_AGG = {F: _make_agg(F, nbuf=(2 if F == 128 else 4)) for F in (16, 32, 64, 128)}


_INV_SQRT2 = 0.7071067811865476


def _gelu(x):
    return 0.5 * x * (1.0 + lax.erf(x * _INV_SQRT2))


def _bn(u, g, bt):
    m = jnp.mean(u, axis=0, keepdims=True)
    d = u - m
    v = jnp.mean(d * d, axis=0, keepdims=True)
    return d * lax.rsqrt(v + 1e-5) * g + bt


def _psum(p_ref):
    return p_ref[0, :_N, :] + p_ref[1, :_N, :]


def _dis_body(p_ref, o_ref):
    cnt = p_ref[0, :_N, 0:1] + p_ref[1, :_N, 0:1]
    o_ref[...] = lax.rsqrt(cnt + 1.0)


def _tc1_body(x_ref, w_ref, dis_ref, o_ref):
    t = jnp.dot(x_ref[...], w_ref[...], preferred_element_type=jnp.float32)
    o_ref[...] = t * dis_ref[...]


def _tc2_body(p_ref, s_ref, dis_ref, b_ref, g_ref, bt_ref, o_ref):
    a = (_psum(p_ref) + s_ref[...]) * dis_ref[...]
    h = _gelu(_bn(a + b_ref[...], g_ref[...], bt_ref[...]))
    o_ref[...] = h * dis_ref[...]


def _tcm_body(p_ref, s_ref, dis_ref, w_ref, b_ref, g_ref, bt_ref, o_ref):
    a = (_psum(p_ref) + s_ref[...]) * dis_ref[...]
    k = w_ref.shape[0]
    u = jnp.dot(a[:, :k], w_ref[...],
                preferred_element_type=jnp.float32) + b_ref[...]
    h = _gelu(_bn(u, g_ref[...], bt_ref[...]))
    o_ref[...] = h * dis_ref[...]


def _tc6_body(p_ref, s_ref, dis_ref, w_ref, b_ref, g_ref, bt_ref,
              oa_ref, ob_ref):
    a = (_psum(p_ref) + s_ref[...]) * dis_ref[...]
    u = jnp.dot(a, w_ref[...], preferred_element_type=jnp.float32) + b_ref[...]
    h = _gelu(_bn(u, g_ref[...], bt_ref[...])) * dis_ref[...]
    oa_ref[...] = h[:, :128]
    ob_ref[...] = h[:, 128:]


_RB = 2000


def _comb6_body(pa_ref, pb_ref, sa_ref, sb_ref, dis_ref, o_ref):
    aa = pa_ref[0] + pa_ref[1] + sa_ref[...]
    ab = pb_ref[0] + pb_ref[1] + sb_ref[...]
    o_ref[...] = jnp.concatenate([aa, ab], axis=1) * dis_ref[...]


def _comb6(pa, pb, sa, sb, dis):
    return pl.pallas_call(
        _comb6_body,
        grid=(_N // _RB,),
        in_specs=[
            pl.BlockSpec((2, _RB, 128), lambda i: (0, i, 0)),
            pl.BlockSpec((2, _RB, 128), lambda i: (0, i, 0)),
            pl.BlockSpec((_RB, 128), lambda i: (i, 0)),
            pl.BlockSpec((_RB, 128), lambda i: (i, 0)),
            pl.BlockSpec((_RB, 1), lambda i: (i, 0)),
        ],
        out_specs=pl.BlockSpec((_RB, 256), lambda i: (i, 0)),
        out_shape=jax.ShapeDtypeStruct((_N, 256), jnp.float32),
    )(pa, pb, sa, sb, dis)


def _tc7_body(a_ref, dis_ref, w6_ref, b6_ref, g6_ref, bt6_ref, w7_ref, o_ref):
    j = pl.program_id(0)
    u = jnp.dot(a_ref[...], w6_ref[...],
                preferred_element_type=jnp.float32) + b6_ref[...]
    h = _gelu(_bn(u, g6_ref[...], bt6_ref[...]))
    pt = jnp.dot(h, w7_ref[...], preferred_element_type=jnp.float32)

    @pl.when(j == 0)
    def _():
        o_ref[...] = pt

    @pl.when(j > 0)
    def _():
        o_ref[...] = o_ref[...] + pt

    @pl.when(j == 3)
    def _():
        o_ref[...] = o_ref[...] * dis_ref[...]


def _tc7(a6, dis, w6, b6, g6, bt6, w7p):
    return pl.pallas_call(
        _tc7_body,
        grid=(4,),
        in_specs=[
            pl.BlockSpec((_N, 256), lambda j: (0, 0)),
            pl.BlockSpec((_N, 1), lambda j: (0, 0)),
            pl.BlockSpec((256, 128), lambda j: (0, j)),
            pl.BlockSpec((1, 128), lambda j: (0, j)),
            pl.BlockSpec((1, 128), lambda j: (0, j)),
            pl.BlockSpec((1, 128), lambda j: (0, j)),
            pl.BlockSpec((128, 64), lambda j: (j, 0)),
        ],
        out_specs=pl.BlockSpec((_N, 64), lambda j: (0, 0)),
        out_shape=jax.ShapeDtypeStruct((_N, 64), jnp.float32),
    )(a6, dis, w6, b6, g6, bt6, w7p)


def _tc8_body(p_ref, s_ref, dis_ref, b_ref, o_ref):
    a = (_psum(p_ref) + s_ref[...]) * dis_ref[...]
    u = a[:, :40] + b_ref[...]
    mx = jnp.max(u, axis=1, keepdims=True)
    sh = u - mx
    lse = jnp.log(jnp.sum(jnp.exp(sh), axis=1, keepdims=True))
    o_ref[...] = sh - lse


def _tc(body, out_shape):
    return pl.pallas_call(body, out_shape=out_shape)


def kernel(x, edge_index, W1, b1, W2, b2, W3, b3, W4, b4, W5, b5, W6, b6,
           W7, b7, g1, bt1, g2, bt2, g3, bt3, g4, bt4, g5, bt5, g6, bt6):
    f32 = jnp.float32
    src = edge_index[0].astype(jnp.int32)
    dst = edge_index[1].astype(jnp.int32)
    pad = _EPAD - _E
    src_p = jnp.concatenate([src, jnp.zeros((pad,), jnp.int32)])
    dst_p = jnp.concatenate([dst, jnp.full((pad,), _N, jnp.int32)])
    idx_p = jnp.stack([src_p.reshape(_TCHUNKS, _CHUNK),
                       dst_p.reshape(_TCHUNKS, _CHUNK)], axis=1)

    ones16 = jnp.ones((_N, 16), f32)

    def agg(h, F):
        return _AGG[F](h, idx_p)

    row = lambda v: v.reshape(1, -1)

    p0 = agg(ones16, 16)
    dis = _tc(_dis_body, jax.ShapeDtypeStruct((_N, 1), f32))(p0)

    s1 = _tc(_tc1_body, jax.ShapeDtypeStruct((_N, 16), f32))(x, W1, dis)
    p1 = agg(s1, 16)
    s2 = _tc(_tc2_body, jax.ShapeDtypeStruct((_N, 16), f32))(
        p1, s1, dis, row(b1), row(g1), row(bt1))
    p2 = agg(s2, 16)
    s3 = _tc(_tcm_body, jax.ShapeDtypeStruct((_N, 32), f32))(
        p2, s2, dis, W2, row(b2), row(g2), row(bt2))
    p3 = agg(s3, 32)
    s4 = _tc(_tcm_body, jax.ShapeDtypeStruct((_N, 64), f32))(
        p3, s3, dis, W3, row(b3), row(g3), row(bt3))
    p4 = agg(s4, 64)
    s5 = _tc(_tcm_body, jax.ShapeDtypeStruct((_N, 128), f32))(
        p4, s4, dis, W4, row(b4), row(g4), row(bt4))
    p5 = agg(s5, 128)
    s6a, s6b = _tc(_tc6_body, (jax.ShapeDtypeStruct((_N, 128), f32),
                               jax.ShapeDtypeStruct((_N, 128), f32)))(
        p5, s5, dis, W5, row(b5), row(g5), row(bt5))
    p6a = _AGG[128](s6a, idx_p)
    p6b = _AGG[128](s6b, idx_p)
    w7p = jnp.concatenate([W7, jnp.zeros((512, 24), f32)], axis=1)
    a6 = _comb6(p6a, p6b, s6a, s6b, dis)
    s7 = _tc7(a6, dis, W6, row(b6), row(g6), row(bt6), w7p)
    p7 = agg(s7, 64)
    out = _tc(_tc8_body, jax.ShapeDtypeStruct((_N, 40), f32))(
        p7, s7, dis, row(b7))
    return out
